# baseline (speedup 1.0000x reference)
import functools

import jax
import jax.numpy as jnp
from jax import lax
from jax.experimental import pallas as pl
from jax.experimental.pallas import tpu as pltpu
from jax.experimental.pallas import tpu_sc as plsc

B, S, D, E = 4, 2048, 4096, 64

NC, NSUB, LANES = 2, 16, 16
NW = NC * NSUB
DCH = D // (NW // B)
ROWS = 64
NCHUNK = S // ROWS

_mesh = plsc.VectorSubcoreMesh(core_axis_name="c", subcore_axis_name="s")


@functools.partial(
    pl.kernel,
    mesh=_mesh,
    out_type=jax.ShapeDtypeStruct((B, D), jnp.float32),
    scratch_types=[
        pltpu.VMEM((ROWS, DCH), jnp.float32),
        pltpu.VMEM((ROWS, DCH), jnp.float32),
        pltpu.VMEM((DCH,), jnp.float32),
        pltpu.SemaphoreType.DMA,
        pltpu.SemaphoreType.DMA,
    ],
)
def _pool_sc(h_hbm, out_hbm, buf0, buf1, acc, sem0, sem1):
    wid = lax.axis_index("s") * NC + lax.axis_index("c")
    b = wid // (D // DCH)
    d0 = (wid % (D // DCH)) * DCH

    for j in range(DCH // LANES):
        acc[pl.ds(j * LANES, LANES)] = jnp.zeros((LANES,), jnp.float32)

    def _copy(g, buf, sem):
        return pltpu.make_async_copy(
            h_hbm.at[b, pl.ds(g * ROWS, ROWS), pl.ds(d0, DCH)], buf, sem)

    def _accum(buf):
        def _row(r, carry):
            for j in range(DCH // LANES):
                sl = pl.ds(j * LANES, LANES)
                plsc.addupdate(acc.at[sl], buf[r, sl])
            return carry
        lax.fori_loop(0, ROWS, _row, 0)

    _copy(0, buf0, sem0).start()

    def _body(i, carry):
        g = 2 * i
        _copy(g + 1, buf1, sem1).start()
        _copy(g, buf0, sem0).wait()
        _accum(buf0)

        @pl.when(g + 2 < NCHUNK)
        def _():
            _copy(g + 2, buf0, sem0).start()

        _copy(g + 1, buf1, sem1).wait()
        _accum(buf1)
        return carry

    lax.fori_loop(0, NCHUNK // 2, _body, 0)
    pltpu.sync_copy(acc, out_hbm.at[b, pl.ds(d0, DCH)])


def _finish_body(sums_ref, w_ref, oh_ref, logits_ref, probs_ref, loss_ref):
    pooled = sums_ref[...] * (1.0 / S)
    logits = jax.lax.dot_general(
        pooled, w_ref[...], (((1,), (1,)), ((), ())),
        preferred_element_type=jnp.float32)
    m = jnp.max(logits, axis=1, keepdims=True)
    ex = jnp.exp(logits - m)
    se = jnp.sum(ex, axis=1, keepdims=True)
    logits_ref[...] = logits
    probs_ref[...] = ex / se
    lse = jnp.log(se) + m
    picked = jnp.sum(logits * oh_ref[...], axis=1, keepdims=True)
    loss_ref[...] = jnp.mean(lse - picked).reshape(1, 1)


@jax.jit
def kernel(hidden_states, W, task_labels):
    onehot = (task_labels[:, None] == jnp.arange(E, dtype=jnp.int32)[None, :])
    onehot = onehot.astype(jnp.float32)
    sums = _pool_sc(hidden_states)
    logits, probs, loss = pl.pallas_call(
        _finish_body,
        out_shape=[
            jax.ShapeDtypeStruct((B, E), jnp.float32),
            jax.ShapeDtypeStruct((B, E), jnp.float32),
            jax.ShapeDtypeStruct((1, 1), jnp.float32),
        ],
    )(sums, W, onehot)
    return logits, probs, loss.reshape(())
